# baseline (speedup 1.0000x reference)
import functools

import jax
import jax.numpy as jnp
from jax import lax
from jax.experimental import pallas as pl
from jax.experimental.pallas import tpu as pltpu
from jax.experimental.pallas import tpu_sc as plsc

NUM_EMB = 1000000
D = 64
BATCH = 16384
HIST = 50
IW = 128
NC, NS = 2, 16
NW = NC * NS
B_PER_W = BATCH // NW
JB = B_PER_W // IW
NU = HIST * JB

_mesh = plsc.VectorSubcoreMesh(
    core_axis_name="c", subcore_axis_name="s", num_cores=NC, num_subcores=NS
)


@functools.partial(
    pl.kernel,
    out_type=jax.ShapeDtypeStruct((HIST, D // 8, BATCH // 128, 8, 128),
                                  jnp.float32),
    mesh=_mesh,
    scratch_types=[
        pltpu.VMEM((HIST, B_PER_W), jnp.int32),
        pltpu.VMEM((IW, D), jnp.float32),
        pltpu.VMEM((IW, D), jnp.float32),
        pltpu.VMEM((IW, D), jnp.float32),
        pltpu.VMEM((IW, D), jnp.float32),
        pltpu.VMEM((D // 8, 8, IW), jnp.float32),
        pltpu.VMEM((D // 8, 8, IW), jnp.float32),
        pltpu.SemaphoreType.DMA,
        pltpu.SemaphoreType.DMA,
        pltpu.SemaphoreType.DMA,
        pltpu.SemaphoreType.DMA,
        pltpu.SemaphoreType.DMA,
        pltpu.SemaphoreType.DMA,
    ],
    compiler_params=pltpu.CompilerParams(
        use_tc_tiling_on_sc=False, needs_layout_passes=False
    ),
)
def _gather_kernel(idx_hbm, table_hbm, out_hbm, idx_v, a0, a1, a2, a3,
                   b0, b1, ga0, ga1, ga2, ga3, sb0, sb1):
    wid = lax.axis_index("s") * NC + lax.axis_index("c")
    col0 = wid * B_PER_W

    pltpu.sync_copy(idx_hbm.at[:, pl.ds(col0, B_PER_W)], idx_v)

    def fire_gather(u, buf, sem):
        h = lax.shift_right_logical(u, 2)
        j = lax.bitwise_and(u, 3)
        pltpu.async_copy(
            table_hbm.at[idx_v.at[h, pl.ds(j * IW, IW)]], buf, sem
        )

    def wait_gather(buf, sem):
        pltpu.make_async_copy(table_hbm.at[pl.ds(0, IW)], buf, sem).wait()

    lanes = jnp.arange(16, dtype=jnp.int32)
    row_vecs = [lanes + rb * 16 for rb in range(IW // 16)]

    def transpose(src, dst):
        @plsc.parallel_loop(0, D, unroll=4)
        def _d(d):
            cols = jnp.zeros((16,), jnp.int32) + d
            rt = lax.shift_right_logical(d, 3)
            rs = lax.bitwise_and(d, 7)
            for rb in range(IW // 16):
                v = plsc.load_gather(src, [row_vecs[rb], cols])
                dst[rt, rs, pl.ds(rb * 16, 16)] = v

    def fire_store(u, buf, sem):
        h = lax.shift_right_logical(u, 2)
        j = lax.bitwise_and(u, 3)
        ct = wid * JB + j
        pltpu.async_copy(
            buf, out_hbm.at[h, :, ct], sem
        )

    def wait_store(buf, sem):
        pltpu.make_async_copy(buf, out_hbm.at[0, :, 0], sem).wait()

    abufs = (a0, a1, a2, a3)
    asems = (ga0, ga1, ga2, ga3)
    bbufs = (b0, b1)
    bsems = (sb0, sb1)

    for i in range(4):
        fire_gather(jnp.int32(i), abufs[i], asems[i])

    @pl.loop(0, NU, step=4)
    def _quad(u):
        for i in range(4):
            wait_gather(abufs[i], asems[i])

            @pl.when(u + i >= 2)
            def _():
                wait_store(bbufs[i % 2], bsems[i % 2])

            transpose(abufs[i], bbufs[i % 2])
            fire_store(u + i, bbufs[i % 2], bsems[i % 2])

            @pl.when(u + i + 4 < NU)
            def _():
                fire_gather(u + i + 4, abufs[i], asems[i])

    wait_store(b0, sb0)
    wait_store(b1, sb1)


def kernel(idx, table):
    idx_t = idx.T * 2
    tpad = jnp.pad(table, ((0, 0), (0, D))).reshape(2 * NUM_EMB, D)
    out5 = _gather_kernel(idx_t, tpad)
    out = out5.transpose(2, 4, 0, 1, 3).reshape(BATCH, HIST, D)
    return out


# speedup vs baseline: 1.7373x; 1.7373x over previous
import functools

import jax
import jax.numpy as jnp
from jax import lax
from jax.experimental import pallas as pl
from jax.experimental.pallas import tpu as pltpu
from jax.experimental.pallas import tpu_sc as plsc

NUM_EMB = 1000000
D = 64
BATCH = 16384
HIST = 50
IW = 128
NC, NS = 2, 16
NW = NC * NS
B_PER_W = BATCH // NW
JB = B_PER_W // IW
NU = HIST * JB

_mesh = plsc.VectorSubcoreMesh(
    core_axis_name="c", subcore_axis_name="s", num_cores=NC, num_subcores=NS
)


@functools.partial(
    pl.kernel,
    out_type=jax.ShapeDtypeStruct((HIST, D // 8, BATCH // 128, 8, 128),
                                  jnp.float32),
    mesh=_mesh,
    scratch_types=[
        pltpu.VMEM((HIST, B_PER_W), jnp.int32),
        pltpu.VMEM((IW, D), jnp.float32),
        pltpu.VMEM((IW, D), jnp.float32),
        pltpu.VMEM((IW, D), jnp.float32),
        pltpu.VMEM((IW, D), jnp.float32),
        pltpu.VMEM((D // 8, 8, IW), jnp.float32),
        pltpu.VMEM((D // 8, 8, IW), jnp.float32),
        pltpu.SemaphoreType.DMA,
        pltpu.SemaphoreType.DMA,
        pltpu.SemaphoreType.DMA,
        pltpu.SemaphoreType.DMA,
        pltpu.SemaphoreType.DMA,
        pltpu.SemaphoreType.DMA,
    ],
    compiler_params=pltpu.CompilerParams(
        use_tc_tiling_on_sc=False, needs_layout_passes=False
    ),
)
def _gather_kernel(idx_hbm, table_hbm, out_hbm, idx_v, a0, a1, a2, a3,
                   b0, b1, ga0, ga1, ga2, ga3, sb0, sb1):
    wid = lax.axis_index("s") * NC + lax.axis_index("c")
    col0 = wid * B_PER_W

    pltpu.sync_copy(idx_hbm.at[:, pl.ds(col0, B_PER_W)], idx_v)

    def fire_gather(u, buf, sem):
        h = lax.shift_right_logical(u, 2)
        j = lax.bitwise_and(u, 3)
        pltpu.async_copy(
            table_hbm.at[idx_v.at[h, pl.ds(j * IW, IW)]], buf, sem
        )

    def wait_gather(buf, sem):
        pltpu.make_async_copy(table_hbm.at[pl.ds(0, IW)], buf, sem).wait()

    lanes = jnp.arange(16, dtype=jnp.int32)
    zeros16 = jnp.zeros((16,), jnp.int32)
    diag_src = [lanes * D + ((lanes + k) & 15) for k in range(16)]
    diag_dst = [((lanes + k) & 15) * IW + lanes for k in range(16)]

    def transpose(src, dst):
        @plsc.parallel_loop(0, (IW // 16) * (D // 16), unroll=2)
        def _blk(t):
            rb = lax.shift_right_logical(t, 2)
            cb = lax.bitwise_and(t, 3)
            soff = rb * (16 * D) + cb * 16
            doff = cb * (16 * IW) + rb * 16
            for k in range(16):
                v = plsc.load_gather(src, [zeros16, diag_src[k] + soff])
                plsc.store_scatter(dst, [zeros16, zeros16, diag_dst[k] + doff], v)

    def fire_store(u, buf, sem):
        h = lax.shift_right_logical(u, 2)
        j = lax.bitwise_and(u, 3)
        ct = wid * JB + j
        pltpu.async_copy(
            buf, out_hbm.at[h, :, ct], sem
        )

    def wait_store(buf, sem):
        pltpu.make_async_copy(buf, out_hbm.at[0, :, 0], sem).wait()

    abufs = (a0, a1, a2, a3)
    asems = (ga0, ga1, ga2, ga3)
    bbufs = (b0, b1)
    bsems = (sb0, sb1)

    for i in range(4):
        fire_gather(jnp.int32(i), abufs[i], asems[i])

    @pl.loop(0, NU, step=4)
    def _quad(u):
        for i in range(4):
            wait_gather(abufs[i], asems[i])

            @pl.when(u + i >= 2)
            def _():
                wait_store(bbufs[i % 2], bsems[i % 2])

            transpose(abufs[i], bbufs[i % 2])
            fire_store(u + i, bbufs[i % 2], bsems[i % 2])

            @pl.when(u + i + 4 < NU)
            def _():
                fire_gather(u + i + 4, abufs[i], asems[i])

    wait_store(b0, sb0)
    wait_store(b1, sb1)


def kernel(idx, table):
    idx_t = idx.T * 2
    tpad = jnp.pad(table, ((0, 0), (0, D))).reshape(2 * NUM_EMB, D)
    out5 = _gather_kernel(idx_t, tpad)
    out = out5.transpose(2, 4, 0, 1, 3).reshape(BATCH, HIST, D)
    return out
